# baseline (speedup 1.0000x reference)
import functools

import jax
import jax.numpy as jnp
from jax import lax
from jax.experimental import pallas as pl
from jax.experimental.pallas import tpu as pltpu
from jax.experimental.pallas import tpu_sc as plsc

B = 16384
F = 26
V = 100000
D = 16

NC = 2
NS = 16
NW = NC * NS
ROWS_W = B // NW
CHUNK_R = 4
N_CHUNK = ROWS_W // CHUNK_R
G_CHUNK = CHUNK_R * F
SUB = 104
N_SUB = G_CHUNK // SUB
NG = F * V // 8

_IOTA = None


def _sc_bi_kernel(idx_hbm, tab_hbm, bi_hbm, psum_hbm, psq_hbm,
                  idx_v, idx_g, rows0, rows1, bi_v, part_v, sem0, sem1):
    wid = lax.axis_index("s") * NC + lax.axis_index("c")
    base = wid * ROWS_W

    pltpu.sync_copy(idx_hbm.at[pl.ds(base * F, ROWS_W * F)],
                    idx_v.at[pl.ds(0, ROWS_W * F)])

    @pl.loop(0, (ROWS_W * F) // 16)
    def _split(k):
        v = idx_v[pl.ds(k * 16, 16)]
        idx_g[pl.ds(k * 16, 16)] = lax.shift_right_logical(v, 3)
        idx_v[pl.ds(k * 16, 16)] = lax.shift_left(
            jnp.bitwise_and(v, 7), 4)

    rows = (rows0, rows1)
    sems = (sem0, sem1)
    iota = lax.iota(jnp.int32, 16)

    def fire(cc, slot):
        for s in range(N_SUB):
            off = cc * G_CHUNK + s * SUB
            pltpu.async_copy(
                tab_hbm.at[idx_g.at[pl.ds(off, SUB)]],
                rows[slot].at[pl.ds(s * SUB, SUB)],
                sems[slot])

    def wait(slot):
        pltpu.make_async_copy(
            tab_hbm.at[idx_g.at[pl.ds(0, G_CHUNK)]],
            rows[slot], sems[slot]).wait()

    def compute(cc, slot, carry):
        ps, pq = carry
        buf = rows[slot]
        coff = cc * G_CHUNK
        s_acc = [None] * CHUNK_R
        q_acc = [None] * CHUNK_R
        sub_blk = None
        for p in range(G_CHUNK):
            j = p % 16
            if j == 0:
                sub_blk = idx_v[pl.ds(coff + p, 16)]
            col = sub_blk.at[jnp.full((16,), j, jnp.int32)].get(
                mode="promise_in_bounds") + iota
            e = plsc.load_gather(buf, [jnp.full((16,), p, jnp.int32), col])
            r = p // F
            if p % F == 0:
                s_acc[r] = e
                q_acc[r] = e * e
            else:
                s_acc[r] = s_acc[r] + e
                q_acc[r] = q_acc[r] + e * e
        for r in range(CHUNK_R):
            bi = 0.5 * (s_acc[r] * s_acc[r] - q_acc[r])
            bi_v[cc * CHUNK_R + r] = bi
            ps = ps + bi
            pq = pq + bi * bi
        return ps, pq

    zeros = jnp.zeros((D,), jnp.float32)
    fire(0, 0)
    fire(1, 1)

    @pl.loop(0, N_CHUNK, step=2, init_carry=(zeros, zeros))
    def _chunks(c, carry):
        for b in range(2):
            wait(b)
            carry = compute(c + b, b, carry)
            fire(jnp.minimum(c + b + 2, N_CHUNK - 1), b)
        return carry

    psum, psq = _chunks
    wait(0)
    wait(1)

    part_v[0] = psum
    part_v[1] = psq
    pltpu.sync_copy(bi_v, bi_hbm.at[pl.ds(base, ROWS_W)])
    pltpu.sync_copy(part_v.at[0], psum_hbm.at[wid])
    pltpu.sync_copy(part_v.at[1], psq_hbm.at[wid])


_sc_bi = functools.partial(
    pl.kernel,
    out_type=[
        jax.ShapeDtypeStruct((B, D), jnp.float32),
        jax.ShapeDtypeStruct((NW, D), jnp.float32),
        jax.ShapeDtypeStruct((NW, D), jnp.float32),
    ],
    mesh=plsc.VectorSubcoreMesh(core_axis_name="c", subcore_axis_name="s"),
    scratch_types=[
        pltpu.VMEM((ROWS_W * F + 16,), jnp.int32),
        pltpu.VMEM((ROWS_W * F,), jnp.int32),
        pltpu.VMEM((G_CHUNK, 128), jnp.float32),
        pltpu.VMEM((G_CHUNK, 128), jnp.float32),
        pltpu.VMEM((ROWS_W, D), jnp.float32),
        pltpu.VMEM((2, D), jnp.float32),
        pltpu.SemaphoreType.DMA,
        pltpu.SemaphoreType.DMA,
    ],
    compiler_params=pltpu.CompilerParams(use_tc_tiling_on_sc=True,
                                         needs_layout_passes=False),
)(_sc_bi_kernel)


def _tc_mlp_kernel(bi_ref, psum_ref, psq_ref, gamma_ref, beta_ref,
                   w1_ref, b1_ref, w2_ref, b2_ref, w3_ref, b3_ref, out_ref):
    inv_b = 1.0 / B
    mean = jnp.sum(psum_ref[...], axis=0, keepdims=True) * inv_b
    ex2 = jnp.sum(psq_ref[...], axis=0, keepdims=True) * inv_b
    var = ex2 - mean * mean
    scale = gamma_ref[...] * jax.lax.rsqrt(var + 1e-3)
    shift = beta_ref[...] - mean * scale
    x = bi_ref[...] * scale + shift
    h = jnp.dot(x, w1_ref[...], preferred_element_type=jnp.float32) + b1_ref[...]
    h = jnp.maximum(h, 0.0)
    h = jnp.dot(h, w2_ref[...], preferred_element_type=jnp.float32) + b2_ref[...]
    h = jnp.maximum(h, 0.0)
    o = jnp.dot(h, w3_ref[...], preferred_element_type=jnp.float32) + b3_ref[...]
    out_ref[...] = 1.0 / (1.0 + jnp.exp(-o))


def kernel(tables, gamma, beta, W1, b1, W2, b2, W3, b3, indices):
    tab_g = tables.reshape(F * V // 8, 8 * D)
    flat_idx = (indices.astype(jnp.int32)
                + (jnp.arange(F, dtype=jnp.int32) * V)[None, :]).reshape(B * F)

    bi, psum, psq = _sc_bi(flat_idx, tab_g)

    out = pl.pallas_call(
        _tc_mlp_kernel,
        out_shape=jax.ShapeDtypeStruct((B, 1), jnp.float32),
    )(bi, psum, psq,
      gamma.reshape(1, D), beta.reshape(1, D),
      W1, b1.reshape(1, 256), W2, b2.reshape(1, 128), W3, b3.reshape(1, 1))
    return out


# speedup vs baseline: 1.3032x; 1.3032x over previous
import functools

import jax
import jax.numpy as jnp
from jax import lax
from jax.experimental import pallas as pl
from jax.experimental.pallas import tpu as pltpu
from jax.experimental.pallas import tpu_sc as plsc

B = 16384
F = 26
V = 100000
D = 16

NC = 2
NS = 16
NW = NC * NS
ROWS_W = B // NW
CHUNK_R = 64
N_CHUNK = ROWS_W // CHUNK_R
G_CHUNK = CHUNK_R * F
SUB = 128
N_SUB = G_CHUNK // SUB

VCH = 4000


def _tc_transpose_kernel(y_ref, out_ref, xt_buf, w0, w1, sem0, sem1):
    f = pl.program_id(0)
    x = y_ref[0]
    bufs = (w0, w1)
    sems = (sem0, sem1)
    descs = [None, None]
    for c in range(V // VCH):
        slot = c % 2
        xt_buf[...] = jnp.transpose(x[:, c * VCH:(c + 1) * VCH], (1, 0))
        w = jnp.concatenate(
            [xt_buf[pl.Slice(s, VCH // 8, 8), :] for s in range(8)], axis=1)
        if descs[slot] is not None:
            descs[slot].wait()
        bufs[slot][...] = w
        descs[slot] = pltpu.async_copy(
            bufs[slot],
            out_ref.at[pl.ds(f * (V // 8) + c * (VCH // 8), VCH // 8)],
            sems[slot])
    descs[0].wait()
    descs[1].wait()


def _relayout_table(tables):
    y = jnp.transpose(tables, (0, 2, 1))
    return pl.pallas_call(
        _tc_transpose_kernel,
        grid=(F,),
        in_specs=[pl.BlockSpec((1, D, V), lambda f: (f, 0, 0))],
        out_specs=pl.BlockSpec(memory_space=pl.ANY),
        out_shape=jax.ShapeDtypeStruct((F * V // 8, 8 * D), jnp.float32),
        scratch_shapes=[
            pltpu.VMEM((VCH, D), jnp.float32),
            pltpu.VMEM((VCH // 8, 8 * D), jnp.float32),
            pltpu.VMEM((VCH // 8, 8 * D), jnp.float32),
            pltpu.SemaphoreType.DMA,
            pltpu.SemaphoreType.DMA,
        ],
    )(y)


def _sc_bi_kernel(idx_hbm, tab_hbm, bi_hbm, psum_hbm, psq_hbm,
                  idx_v, rows0, rows1, bi_v, part_v, sem0, sem1):
    wid = lax.axis_index("s") * NC + lax.axis_index("c")
    base = wid * ROWS_W

    pltpu.sync_copy(idx_hbm.at[pl.ds(base * F, ROWS_W * F)], idx_v)

    rows = (rows0, rows1)
    sems = (sem0, sem1)

    def fire(c, slot):
        descs = []
        for s in range(N_SUB):
            off = c * G_CHUNK + s * SUB
            descs.append(pltpu.async_copy(
                tab_hbm.at[idx_v.at[pl.ds(off, SUB)]],
                rows[slot].at[pl.ds(s * SUB, SUB)],
                sems[slot]))
        return descs

    zeros = jnp.zeros((D,), jnp.float32)
    psum = zeros
    psq = zeros

    inflight = {0: fire(0, 0), 1: None}
    for c in range(N_CHUNK):
        slot = c % 2
        if c + 1 < N_CHUNK:
            inflight[1 - slot] = fire(c + 1, 1 - slot)
        for d_ in inflight[slot]:
            d_.wait()
        buf = rows[slot]

        @pl.loop(0, CHUNK_R, init_carry=(psum, psq))
        def _row(r, carry):
            ps, pq = carry
            e = buf[r * F]
            s = e
            sq = e * e
            for f in range(1, F):
                e = buf[r * F + f]
                s = s + e
                sq = sq + e * e
            bi = 0.5 * (s * s - sq)
            bi_v[c * CHUNK_R + r] = bi
            return ps + bi, pq + bi * bi

        psum, psq = _row

    part_v[0] = psum
    part_v[1] = psq
    pltpu.sync_copy(bi_v, bi_hbm.at[pl.ds(base, ROWS_W)])
    pltpu.sync_copy(part_v.at[0], psum_hbm.at[wid])
    pltpu.sync_copy(part_v.at[1], psq_hbm.at[wid])


_sc_bi = functools.partial(
    pl.kernel,
    out_type=[
        jax.ShapeDtypeStruct((B, D), jnp.float32),
        jax.ShapeDtypeStruct((NW, D), jnp.float32),
        jax.ShapeDtypeStruct((NW, D), jnp.float32),
    ],
    mesh=plsc.VectorSubcoreMesh(core_axis_name="c", subcore_axis_name="s"),
    scratch_types=[
        pltpu.VMEM((ROWS_W * F,), jnp.int32),
        pltpu.VMEM((G_CHUNK, D), jnp.float32),
        pltpu.VMEM((G_CHUNK, D), jnp.float32),
        pltpu.VMEM((ROWS_W, D), jnp.float32),
        pltpu.VMEM((2, D), jnp.float32),
        pltpu.SemaphoreType.DMA,
        pltpu.SemaphoreType.DMA,
    ],
    compiler_params=pltpu.CompilerParams(use_tc_tiling_on_sc=False),
)(_sc_bi_kernel)


def _tc_mlp_kernel(bi_ref, psum_ref, psq_ref, gamma_ref, beta_ref,
                   w1_ref, b1_ref, w2_ref, b2_ref, w3_ref, b3_ref, out_ref):
    inv_b = 1.0 / B
    mean = jnp.sum(psum_ref[...], axis=0, keepdims=True) * inv_b
    ex2 = jnp.sum(psq_ref[...], axis=0, keepdims=True) * inv_b
    var = ex2 - mean * mean
    scale = gamma_ref[...] * jax.lax.rsqrt(var + 1e-3)
    shift = beta_ref[...] - mean * scale
    x = bi_ref[...] * scale + shift
    h = jnp.dot(x, w1_ref[...], preferred_element_type=jnp.float32) + b1_ref[...]
    h = jnp.maximum(h, 0.0)
    h = jnp.dot(h, w2_ref[...], preferred_element_type=jnp.float32) + b2_ref[...]
    h = jnp.maximum(h, 0.0)
    o = jnp.dot(h, w3_ref[...], preferred_element_type=jnp.float32) + b3_ref[...]
    out_ref[...] = 1.0 / (1.0 + jnp.exp(-o))


def kernel(tables, gamma, beta, W1, b1, W2, b2, W3, b3, indices):
    tab_flat = _relayout_table(tables).reshape(F * V, D)
    flat_idx = (indices.astype(jnp.int32)
                + (jnp.arange(F, dtype=jnp.int32) * V)[None, :]).reshape(B * F)

    bi, psum, psq = _sc_bi(flat_idx, tab_flat)

    out = pl.pallas_call(
        _tc_mlp_kernel,
        out_shape=jax.ShapeDtypeStruct((B, 1), jnp.float32),
    )(bi, psum, psq,
      gamma.reshape(1, D), beta.reshape(1, D),
      W1, b1.reshape(1, 256), W2, b2.reshape(1, 128), W3, b3.reshape(1, 1))
    return out


# speedup vs baseline: 5.7769x; 4.4329x over previous
import functools

import jax
import jax.numpy as jnp
from jax import lax
from jax.experimental import pallas as pl
from jax.experimental.pallas import tpu as pltpu
from jax.experimental.pallas import tpu_sc as plsc

B = 16384
F = 26
V = 100000
D = 16

NC = 2
NS = 16
NW = NC * NS
ROWS_W = B // NW
CHUNK_R = 64
N_CHUNK = ROWS_W // CHUNK_R
G_CHUNK = CHUNK_R * F
SUB = 128
N_SUB = G_CHUNK // SUB

VCH = 12800
NVC = 8
NFG = 4
VPAD = NVC * VCH
GROW = 8 * VPAD


def _tc_transpose_kernel(y_ref, out_ref):
    out_ref[...] = jnp.transpose(y_ref[...], (1, 0))


def _relayout_table(tables):
    y = jnp.transpose(tables, (0, 2, 1)).reshape(F * D, V)
    return pl.pallas_call(
        _tc_transpose_kernel,
        grid=(NFG, NVC),
        in_specs=[pl.BlockSpec((128, VCH), lambda g, c: (g, c))],
        out_specs=pl.BlockSpec((VCH, 128), lambda g, c: (g * NVC + c, 0)),
        out_shape=jax.ShapeDtypeStruct((NFG * VPAD, 128), jnp.float32),
    )(y)


def _sc_bi_kernel(idx_hbm, tab_hbm, bi_hbm, psum_hbm, psq_hbm,
                  idx_v, rows0, rows1, bi_v, part_v, sem0, sem1):
    wid = lax.axis_index("s") * NC + lax.axis_index("c")
    base = wid * ROWS_W

    pltpu.sync_copy(idx_hbm.at[pl.ds(base * F, ROWS_W * F)], idx_v)

    rows = (rows0, rows1)
    sems = (sem0, sem1)

    def fire(c, slot):
        descs = []
        for s in range(N_SUB):
            off = c * G_CHUNK + s * SUB
            descs.append(pltpu.async_copy(
                tab_hbm.at[idx_v.at[pl.ds(off, SUB)]],
                rows[slot].at[pl.ds(s * SUB, SUB)],
                sems[slot]))
        return descs

    zeros = jnp.zeros((D,), jnp.float32)
    psum = zeros
    psq = zeros

    inflight = {0: fire(0, 0), 1: None}
    for c in range(N_CHUNK):
        slot = c % 2
        if c + 1 < N_CHUNK:
            inflight[1 - slot] = fire(c + 1, 1 - slot)
        for d_ in inflight[slot]:
            d_.wait()
        buf = rows[slot]

        @pl.loop(0, CHUNK_R, init_carry=(psum, psq))
        def _row(r, carry):
            ps, pq = carry
            e = buf[r * F]
            s = e
            sq = e * e
            for f in range(1, F):
                e = buf[r * F + f]
                s = s + e
                sq = sq + e * e
            bi = 0.5 * (s * s - sq)
            bi_v[c * CHUNK_R + r] = bi
            return ps + bi, pq + bi * bi

        psum, psq = _row

    part_v[0] = psum
    part_v[1] = psq
    pltpu.sync_copy(bi_v, bi_hbm.at[pl.ds(base, ROWS_W)])
    pltpu.sync_copy(part_v.at[0], psum_hbm.at[wid])
    pltpu.sync_copy(part_v.at[1], psq_hbm.at[wid])


_sc_bi = functools.partial(
    pl.kernel,
    out_type=[
        jax.ShapeDtypeStruct((B, D), jnp.float32),
        jax.ShapeDtypeStruct((NW, D), jnp.float32),
        jax.ShapeDtypeStruct((NW, D), jnp.float32),
    ],
    mesh=plsc.VectorSubcoreMesh(core_axis_name="c", subcore_axis_name="s"),
    scratch_types=[
        pltpu.VMEM((ROWS_W * F,), jnp.int32),
        pltpu.VMEM((G_CHUNK, D), jnp.float32),
        pltpu.VMEM((G_CHUNK, D), jnp.float32),
        pltpu.VMEM((ROWS_W, D), jnp.float32),
        pltpu.VMEM((2, D), jnp.float32),
        pltpu.SemaphoreType.DMA,
        pltpu.SemaphoreType.DMA,
    ],
    compiler_params=pltpu.CompilerParams(use_tc_tiling_on_sc=False),
)(_sc_bi_kernel)


def _tc_mlp_kernel(bi_ref, psum_ref, psq_ref, gamma_ref, beta_ref,
                   w1_ref, b1_ref, w2_ref, b2_ref, w3_ref, b3_ref, out_ref):
    inv_b = 1.0 / B
    mean = jnp.sum(psum_ref[...], axis=0, keepdims=True) * inv_b
    ex2 = jnp.sum(psq_ref[...], axis=0, keepdims=True) * inv_b
    var = ex2 - mean * mean
    scale = gamma_ref[...] * jax.lax.rsqrt(var + 1e-3)
    shift = beta_ref[...] - mean * scale
    x = bi_ref[...] * scale + shift
    h = jnp.dot(x, w1_ref[...], preferred_element_type=jnp.float32) + b1_ref[...]
    h = jnp.maximum(h, 0.0)
    h = jnp.dot(h, w2_ref[...], preferred_element_type=jnp.float32) + b2_ref[...]
    h = jnp.maximum(h, 0.0)
    o = jnp.dot(h, w3_ref[...], preferred_element_type=jnp.float32) + b3_ref[...]
    out_ref[...] = 1.0 / (1.0 + jnp.exp(-o))


def kernel(tables, gamma, beta, W1, b1, W2, b2, W3, b3, indices):
    tab_flat = _relayout_table(tables).reshape(NFG * GROW, D)
    karr = jnp.arange(F, dtype=jnp.int32)
    flat_idx = (indices.astype(jnp.int32) * 8
                + ((karr // 8) * GROW + karr % 8)[None, :]).reshape(B * F)

    bi, psum, psq = _sc_bi(flat_idx, tab_flat)

    out = pl.pallas_call(
        _tc_mlp_kernel,
        out_shape=jax.ShapeDtypeStruct((B, 1), jnp.float32),
    )(bi, psum, psq,
      gamma.reshape(1, D), beta.reshape(1, D),
      W1, b1.reshape(1, 256), W2, b2.reshape(1, 128), W3, b3.reshape(1, 1))
    return out


# speedup vs baseline: 6.2538x; 1.0825x over previous
import functools

import jax
import jax.numpy as jnp
from jax import lax
from jax.experimental import pallas as pl
from jax.experimental.pallas import tpu as pltpu
from jax.experimental.pallas import tpu_sc as plsc

B = 16384
F = 26
V = 100000
D = 16

NC = 2
NS = 16
NW = NC * NS
ROWS_W = B // NW
CHUNK_R = 64
N_CHUNK = ROWS_W // CHUNK_R
G_CHUNK = CHUNK_R * F
SUB = 128
N_SUB = G_CHUNK // SUB

VCH = 12800
NVC = 8
NFG = 4
VPAD = NVC * VCH
GROW = 8 * VPAD


def _tc_transpose_kernel(y_ref, out_ref):
    out_ref[...] = jnp.transpose(y_ref[...], (1, 0))


def _relayout_table(tables):
    y = jnp.transpose(tables, (0, 2, 1)).reshape(F * D, V)
    return pl.pallas_call(
        _tc_transpose_kernel,
        grid=(NFG, NVC),
        in_specs=[pl.BlockSpec((128, VCH), lambda g, c: (g, c))],
        out_specs=pl.BlockSpec((VCH, 128), lambda g, c: (g * NVC + c, 0)),
        out_shape=jax.ShapeDtypeStruct((NFG * VPAD, 128), jnp.float32),
    )(y)


def _sc_bi_kernel(idx_hbm, tab_hbm, bi_hbm, psum_hbm, psq_hbm,
                  idx_v, rows0, rows1, bi_v, part_v, sem0, sem1):
    wid = lax.axis_index("s") * NC + lax.axis_index("c")
    base = wid * ROWS_W

    stage = [pltpu.async_copy(
        idx_hbm.at[pl.ds(f * B + base, ROWS_W)],
        idx_v.at[pl.ds(f * ROWS_W, ROWS_W)], sem0) for f in range(F)]
    for d_ in stage:
        d_.wait()

    rows = (rows0, rows1)
    sems = (sem0, sem1)

    def fire(c, slot):
        descs = []
        for f in range(F):
            descs.append(pltpu.async_copy(
                tab_hbm.at[idx_v.at[pl.ds(f * ROWS_W + c * CHUNK_R, CHUNK_R)]],
                rows[slot].at[pl.ds(f * CHUNK_R, CHUNK_R)],
                sems[slot]))
        return descs

    zeros = jnp.zeros((D,), jnp.float32)
    psum = zeros
    psq = zeros

    inflight = {0: fire(0, 0), 1: None}
    for c in range(N_CHUNK):
        slot = c % 2
        if c + 1 < N_CHUNK:
            inflight[1 - slot] = fire(c + 1, 1 - slot)
        for d_ in inflight[slot]:
            d_.wait()
        buf = rows[slot]

        @pl.loop(0, CHUNK_R, init_carry=(psum, psq), unroll=2)
        def _row(r, carry):
            ps, pq = carry
            e = buf[r]
            s = e
            sq = e * e
            for f in range(1, F):
                e = buf[f * CHUNK_R + r]
                s = s + e
                sq = sq + e * e
            bi = 0.5 * (s * s - sq)
            bi_v[c * CHUNK_R + r] = bi
            return ps + bi, pq + bi * bi

        psum, psq = _row

    part_v[0] = psum
    part_v[1] = psq
    pltpu.sync_copy(bi_v, bi_hbm.at[pl.ds(base, ROWS_W)])
    pltpu.sync_copy(part_v.at[0], psum_hbm.at[wid])
    pltpu.sync_copy(part_v.at[1], psq_hbm.at[wid])


_sc_bi = functools.partial(
    pl.kernel,
    out_type=[
        jax.ShapeDtypeStruct((B, D), jnp.float32),
        jax.ShapeDtypeStruct((NW, D), jnp.float32),
        jax.ShapeDtypeStruct((NW, D), jnp.float32),
    ],
    mesh=plsc.VectorSubcoreMesh(core_axis_name="c", subcore_axis_name="s"),
    scratch_types=[
        pltpu.VMEM((ROWS_W * F,), jnp.int32),
        pltpu.VMEM((G_CHUNK, D), jnp.float32),
        pltpu.VMEM((G_CHUNK, D), jnp.float32),
        pltpu.VMEM((ROWS_W, D), jnp.float32),
        pltpu.VMEM((2, D), jnp.float32),
        pltpu.SemaphoreType.DMA,
        pltpu.SemaphoreType.DMA,
    ],
    compiler_params=pltpu.CompilerParams(use_tc_tiling_on_sc=False),
)(_sc_bi_kernel)


def _tc_mlp_kernel(bi_ref, psum_ref, psq_ref, gamma_ref, beta_ref,
                   w1_ref, b1_ref, w2_ref, b2_ref, w3_ref, b3_ref, out_ref):
    inv_b = 1.0 / B
    mean = jnp.sum(psum_ref[...], axis=0, keepdims=True) * inv_b
    ex2 = jnp.sum(psq_ref[...], axis=0, keepdims=True) * inv_b
    var = ex2 - mean * mean
    scale = gamma_ref[...] * jax.lax.rsqrt(var + 1e-3)
    shift = beta_ref[...] - mean * scale
    x = bi_ref[...] * scale + shift
    h = jnp.dot(x, w1_ref[...], preferred_element_type=jnp.float32) + b1_ref[...]
    h = jnp.maximum(h, 0.0)
    h = jnp.dot(h, w2_ref[...], preferred_element_type=jnp.float32) + b2_ref[...]
    h = jnp.maximum(h, 0.0)
    o = jnp.dot(h, w3_ref[...], preferred_element_type=jnp.float32) + b3_ref[...]
    out_ref[...] = 1.0 / (1.0 + jnp.exp(-o))


def kernel(tables, gamma, beta, W1, b1, W2, b2, W3, b3, indices):
    tab_flat = _relayout_table(tables).reshape(NFG * GROW, D)
    karr = jnp.arange(F, dtype=jnp.int32)
    flat_idx = (indices.astype(jnp.int32).T * 8
                + ((karr // 8) * GROW + karr % 8)[:, None]).reshape(F * B)

    bi, psum, psq = _sc_bi(flat_idx, tab_flat)

    out = pl.pallas_call(
        _tc_mlp_kernel,
        out_shape=jax.ShapeDtypeStruct((B, 1), jnp.float32),
    )(bi, psum, psq,
      gamma.reshape(1, D), beta.reshape(1, D),
      W1, b1.reshape(1, 256), W2, b2.reshape(1, 128), W3, b3.reshape(1, 1))
    return out
